# baseline (speedup 1.0000x reference)
import math

import numpy as np
import jax
import jax.numpy as jnp
from jax import lax
from jax.experimental import pallas as pl
from jax.experimental.pallas import tpu as pltpu

_NEG_INF = float("-inf")


def _index_sample_t(l_q: int, l_k: int, sample_k: int):
    idx = jax.random.randint(jax.random.key(42), (l_q, sample_k), 0, l_k)
    return idx.T.astype(jnp.int32)


def _attn_body(n_top, sample_k, n_heads, chunk, idx_ref, mask_ref, q_ref,
               k_ref, v_ref, o_ref, cnt_ref):
    L = q_ref.shape[1]
    D = q_ref.shape[2]
    h = pl.program_id(0)

    @pl.when(h == 0)
    def _build_count():
        for c in range(0, L, chunk):
            row = lax.broadcasted_iota(jnp.int32, (L, chunk), 0)
            acc = jnp.zeros((L, chunk), jnp.float32)
            for s in range(sample_k):
                acc = acc + (row == idx_ref[s:s + 1, c:c + chunk]).astype(
                    jnp.float32)
            cnt_ref[:, c:c + chunk] = acc

    q = q_ref[0]
    k = k_ref[0]
    v = v_ref[0]

    m_parts = []
    for c in range(0, L, chunk):
        st = lax.dot_general(k, q[c:c + chunk, :], (((1,), (1,)), ((), ())),
                             preferred_element_type=jnp.float32)
        cnt = cnt_ref[:, c:c + chunk]
        mmax = jnp.max(jnp.where(cnt > 0.0, st, _NEG_INF), axis=0,
                       keepdims=True)
        msum = jnp.sum(st * cnt, axis=0, keepdims=True)
        m_parts.append(mmax - msum * (1.0 / L))
    m_all = jnp.concatenate(m_parts, axis=1)

    lane = lax.broadcasted_iota(jnp.int32, (1, L), 1)
    sub = lax.broadcasted_iota(jnp.int32, (n_top, 1), 0)
    onehot = jnp.zeros((n_top, L), jnp.float32)
    m_cur = m_all
    for i in range(n_top):
        mx = jnp.max(m_cur, axis=1, keepdims=True)
        idx_i = jnp.min(jnp.where(m_cur == mx, lane, L), axis=1,
                        keepdims=True)
        hit = (lane == idx_i).astype(jnp.float32)
        onehot = onehot + jnp.where(sub == i, hit, 0.0)
        m_cur = jnp.where(lane == idx_i, _NEG_INF, m_cur)

    q_red = jnp.dot(onehot, q, preferred_element_type=jnp.float32)
    scores = lax.dot_general(q_red, k, (((1,), (1,)), ((), ())),
                             preferred_element_type=jnp.float32)
    scores = scores * (1.0 / math.sqrt(D))
    scores = jnp.where(mask_ref[...] == 0, _NEG_INF, scores)
    smx = jnp.max(scores, axis=1, keepdims=True)
    e = jnp.exp(scores - smx)
    a = e / jnp.sum(e, axis=1, keepdims=True)
    o_ref[0] = jnp.dot(a, v, preferred_element_type=jnp.float32)


def kernel(queries, keys, values, attn_mask):
    B, L_Q, H, D = queries.shape
    L_K = keys.shape[1]
    factor = 5
    u_part = int(factor * math.ceil(math.log(max(L_K, 1))))
    u = int(factor * math.ceil(math.log(max(L_Q, 1))))
    u_part = max(min(u_part, L_K), 1)
    u = max(min(u, L_Q), 1)
    sample_k = min(u_part, L_K)
    n_top = min(u, L_Q)

    idx_t = _index_sample_t(L_Q, L_K, sample_k)
    mask_i = attn_mask.astype(jnp.int32)
    chunk = 512

    q_t = jnp.swapaxes(queries, 1, 2).reshape(B * H, L_Q, D)
    k_t = jnp.swapaxes(keys, 1, 2).reshape(B * H, L_K, D)
    v_t = jnp.swapaxes(values, 1, 2).reshape(B * H, L_K, D)

    body = lambda *refs: _attn_body(n_top, sample_k, H, chunk, *refs)
    out = pl.pallas_call(
        body,
        grid=(B * H,),
        in_specs=[
            pl.BlockSpec((sample_k, L_Q), lambda i: (0, 0)),
            pl.BlockSpec((1, L_K), lambda i: (i // H, 0)),
            pl.BlockSpec((1, L_Q, D), lambda i: (i, 0, 0)),
            pl.BlockSpec((1, L_K, D), lambda i: (i, 0, 0)),
            pl.BlockSpec((1, L_K, D), lambda i: (i, 0, 0)),
        ],
        out_specs=pl.BlockSpec((1, n_top, D), lambda i: (i, 0, 0)),
        out_shape=jax.ShapeDtypeStruct((B * H, n_top, D), jnp.float32),
        scratch_shapes=[pltpu.VMEM((L_K, L_Q), jnp.float32)],
    )(idx_t, mask_i, q_t, k_t, v_t)
    return jnp.swapaxes(out.reshape(B, H, n_top, D), 1, 2)


# speedup vs baseline: 1.5504x; 1.5504x over previous
import math

import jax
import jax.numpy as jnp
from jax import lax
from jax.experimental import pallas as pl
from jax.experimental.pallas import tpu as pltpu

_NEG_INF = float("-inf")


def _index_sample_t(l_q: int, l_k: int, sample_k: int):
    idx = jax.random.randint(jax.random.key(42), (l_q, sample_k), 0, l_k)
    return idx.T.astype(jnp.int32)


def _attn_body(n_top, sample_k, n_bh, chunk, idx_ref, mask_ref, q_ref,
               k_ref, v_ref, o_ref, cnt_ref, m_ref, oh_ref):
    L = q_ref.shape[1]
    D = q_ref.shape[2]
    i = pl.program_id(0)

    @pl.when(i == 0)
    def _build_count():
        for c in range(0, L, chunk):
            row = lax.broadcasted_iota(jnp.int32, (L, chunk), 0)
            acc = jnp.zeros((L, chunk), jnp.float32)
            for s in range(sample_k):
                acc = acc + (row == idx_ref[s:s + 1, c:c + chunk]).astype(
                    jnp.float32)
            cnt_ref[:, c:c + chunk] = acc

    @pl.when(i < n_bh)
    def _compute_m():
        q = q_ref[0]
        k = k_ref[0]
        for c in range(0, L, chunk):
            st = lax.dot_general(k, q[c:c + chunk, :],
                                 (((1,), (1,)), ((), ())),
                                 preferred_element_type=jnp.float32)
            cnt = cnt_ref[:, c:c + chunk]
            mmax = jnp.max(jnp.where(cnt > 0.0, st, _NEG_INF), axis=0,
                           keepdims=True)
            msum = jnp.sum(st * cnt, axis=0, keepdims=True)
            m_ref[pl.ds(i, 1), c:c + chunk] = mmax - msum * (1.0 / L)

    @pl.when(i == n_bh)
    def _topk():
        lane = lax.broadcasted_iota(jnp.int32, (n_bh, L), 1)
        m_cur = m_ref[...]
        for r in range(n_top):
            mx = jnp.max(m_cur, axis=1, keepdims=True)
            idx_r = jnp.min(jnp.where(m_cur == mx, lane, L), axis=1,
                            keepdims=True)
            oh_ref[:, r, :] = (lane == idx_r).astype(jnp.float32)
            m_cur = jnp.where(lane == idx_r, _NEG_INF, m_cur)

    @pl.when(i >= n_bh)
    def _attend():
        h = i - n_bh
        q = q_ref[0]
        k = k_ref[0]
        v = v_ref[0]
        onehot = oh_ref[h]
        q_red = jnp.dot(onehot, q, preferred_element_type=jnp.float32)
        scores = lax.dot_general(q_red, k, (((1,), (1,)), ((), ())),
                                 preferred_element_type=jnp.float32)
        scores = scores * (1.0 / math.sqrt(D))
        scores = jnp.where(mask_ref[...] == 0, _NEG_INF, scores)
        smx = jnp.max(scores, axis=1, keepdims=True)
        e = jnp.exp(scores - smx)
        a = e / jnp.sum(e, axis=1, keepdims=True)
        o_ref[0] = jnp.dot(a, v, preferred_element_type=jnp.float32)


def kernel(queries, keys, values, attn_mask):
    B, L_Q, H, D = queries.shape
    L_K = keys.shape[1]
    factor = 5
    u_part = int(factor * math.ceil(math.log(max(L_K, 1))))
    u = int(factor * math.ceil(math.log(max(L_Q, 1))))
    u_part = max(min(u_part, L_K), 1)
    u = max(min(u, L_Q), 1)
    sample_k = min(u_part, L_K)
    n_top = min(u, L_Q)

    idx_t = _index_sample_t(L_Q, L_K, sample_k)
    mask_i = attn_mask.astype(jnp.int32)
    chunk = 512
    BH = B * H

    q_t = jnp.swapaxes(queries, 1, 2).reshape(BH, L_Q, D)
    k_t = jnp.swapaxes(keys, 1, 2).reshape(BH, L_K, D)
    v_t = jnp.swapaxes(values, 1, 2).reshape(BH, L_K, D)

    body = lambda *refs: _attn_body(n_top, sample_k, BH, chunk, *refs)
    out = pl.pallas_call(
        body,
        grid=(2 * BH,),
        in_specs=[
            pl.BlockSpec((sample_k, L_Q), lambda i: (0, 0)),
            pl.BlockSpec((1, L_K), lambda i: ((i % BH) // H, 0)),
            pl.BlockSpec((1, L_Q, D), lambda i: (i % BH, 0, 0)),
            pl.BlockSpec((1, L_K, D), lambda i: (i % BH, 0, 0)),
            pl.BlockSpec((1, L_K, D), lambda i: (i % BH, 0, 0)),
        ],
        out_specs=pl.BlockSpec((1, n_top, D), lambda i: (i % BH, 0, 0)),
        out_shape=jax.ShapeDtypeStruct((BH, n_top, D), jnp.float32),
        scratch_shapes=[
            pltpu.VMEM((L_K, L_Q), jnp.float32),
            pltpu.VMEM((BH, L_Q), jnp.float32),
            pltpu.VMEM((BH, n_top, L_Q), jnp.float32),
        ],
    )(idx_t, mask_i, q_t, k_t, v_t)
    return jnp.swapaxes(out.reshape(B, H, n_top, D), 1, 2)
